# baseline (speedup 1.0000x reference)
import functools

import jax
import jax.numpy as jnp
from jax import lax
from jax.experimental import pallas as pl
from jax.experimental.pallas import tpu as pltpu
from jax.experimental.pallas import tpu_sc as plsc

_NEG_INF = float("-inf")
_I32_MAX = jnp.iinfo(jnp.int32).max


def _top3(cand_v, cand_i, extra_last=0):
    vs, ids = [], []
    for t in range(3):
        m = jnp.max(cand_v, axis=1, keepdims=True)
        sel = jnp.min(jnp.where(cand_v == m, cand_i, _I32_MAX),
                      axis=1, keepdims=True)
        vs.append(m)
        ids.append(sel)
        if t < 2:
            cand_v = jnp.where(cand_i == sel, _NEG_INF, cand_v)
    return vs, ids


def _dist_topk_body(x_ref, e_ref, x2_ref, e2_ref, dist_ref, cv_ref, ci_ref,
                    *, bm, bn, scr):
    j = pl.program_id(1)

    xe = lax.dot_general(
        x_ref[...], e_ref[...],
        dimension_numbers=(((1,), (1,)), ((), ())),
        preferred_element_type=jnp.float32)
    d_tile = -((x2_ref[...] - 2.0 * xe) + e2_ref[...])
    dist_ref[...] = d_tile

    liota = lax.broadcasted_iota(jnp.int32, (1, bn), 1).astype(jnp.float32)
    dd = d_tile
    tvs, tsel = [], []
    for t in range(3):
        m = jnp.max(dd, axis=1, keepdims=True)
        sel = jnp.min(jnp.where(dd == m, liota, jnp.inf),
                      axis=1, keepdims=True)
        tvs.append(m)
        tsel.append(sel)
        if t < 2:
            dd = jnp.where(liota == sel, _NEG_INF, dd)
    tis = [s.astype(jnp.int32) + j * bn for s in tsel]

    pad_v = jnp.full((bm, scr - 3), _NEG_INF, jnp.float32)
    pad_i = jnp.full((bm, scr - 3), _I32_MAX, jnp.int32)
    cv_ref[...] = jnp.concatenate(tvs + [pad_v], axis=1)[None]
    ci_ref[...] = jnp.concatenate(tis + [pad_i], axis=1)[None]


def _merge_body(cv_ref, ci_ref, idx_ref, *, nn, scr):
    cv = jnp.concatenate([cv_ref[jj] for jj in range(nn)], axis=1)
    ci = jnp.concatenate([ci_ref[jj] for jj in range(nn)], axis=1)
    _, mis = _top3(cv, ci)
    bm = cv.shape[0]
    idx_ref[...] = jnp.concatenate(
        mis + [jnp.zeros((bm, scr - 3), jnp.int32)], axis=1)


def _dist_topk(flat, e, x2, e2, *, bm=1024, bn=2048, interpret=False):
    bnrows, d = flat.shape
    kk = e.shape[0]
    nm, nn = bnrows // bm, kk // bn
    scr = 8
    body = functools.partial(_dist_topk_body, bm=bm, bn=bn, scr=scr)
    dist2d, cv, ci = pl.pallas_call(
        body,
        grid=(nm, nn),
        in_specs=[
            pl.BlockSpec((bm, d), lambda i, j: (i, 0)),
            pl.BlockSpec((bn, d), lambda i, j: (j, 0)),
            pl.BlockSpec((bm, 1), lambda i, j: (i, 0)),
            pl.BlockSpec((1, bn), lambda i, j: (0, j)),
        ],
        out_specs=[
            pl.BlockSpec((bm, bn), lambda i, j: (i, j)),
            pl.BlockSpec((1, bm, scr), lambda i, j: (j, i, 0)),
            pl.BlockSpec((1, bm, scr), lambda i, j: (j, i, 0)),
        ],
        out_shape=[
            jax.ShapeDtypeStruct((bnrows, kk), jnp.float32),
            jax.ShapeDtypeStruct((nn, bnrows, scr), jnp.float32),
            jax.ShapeDtypeStruct((nn, bnrows, scr), jnp.int32),
        ],
        interpret=interpret,
    )(flat, e, x2, e2)

    mbody = functools.partial(_merge_body, nn=nn, scr=scr)
    idx3 = pl.pallas_call(
        mbody,
        grid=(nm,),
        in_specs=[
            pl.BlockSpec((nn, bm, scr), lambda i: (0, i, 0)),
            pl.BlockSpec((nn, bm, scr), lambda i: (0, i, 0)),
        ],
        out_specs=pl.BlockSpec((bm, scr), lambda i: (i, 0)),
        out_shape=jax.ShapeDtypeStruct((bnrows, scr), jnp.int32),
        interpret=interpret,
    )(cv, ci)
    return dist2d, idx3


def _sc_gather(table, idx):
    v, d = table.shape
    b = idx.shape[0]
    info = plsc.get_sparse_core_info()
    nc, ns = info.num_cores, info.num_subcores
    nw = nc * ns
    b_per_w = b // nw
    mesh = plsc.VectorSubcoreMesh(core_axis_name="c", subcore_axis_name="s")

    @functools.partial(
        pl.kernel, mesh=mesh,
        out_type=jax.ShapeDtypeStruct((b, d), jnp.float32),
        scratch_types=[
            pltpu.VMEM((b_per_w,), jnp.int32),
            pltpu.VMEM((b_per_w, d), jnp.float32),
            pltpu.SemaphoreType.DMA,
        ],
    )
    def gather_k(table_hbm, idx_hbm, out_hbm, idx_v, rows_v, sem):
        wid = lax.axis_index("s") * nc + lax.axis_index("c")
        base = wid * b_per_w
        pltpu.sync_copy(idx_hbm.at[pl.ds(base, b_per_w)], idx_v)
        pltpu.async_copy(table_hbm.at[idx_v], rows_v, sem).wait()
        pltpu.sync_copy(rows_v, out_hbm.at[pl.ds(base, b_per_w)])

    return gather_k(table, idx)


def kernel(x, k, embed):
    b, n, d = x.shape
    kk = embed.shape[1]
    flat = x.reshape(b * n, d)
    e = embed[0]

    x2 = jnp.sum(flat ** 2, axis=-1, keepdims=True)
    e2 = jnp.sum(embed ** 2, axis=-1)

    dist2d, idx3 = _dist_topk(flat, e, x2, e2)
    ind = jnp.take(idx3[:, :3], k, axis=1)

    quantize = _sc_gather(e, ind).reshape(b, n, d)
    embed_ind = ind.reshape(b, n)
    dist = dist2d.reshape(1, b * n, kk)
    return quantize, embed_ind, dist


# speedup vs baseline: 1.0058x; 1.0058x over previous
import functools

import jax
import jax.numpy as jnp
from jax import lax
from jax.experimental import pallas as pl
from jax.experimental.pallas import tpu as pltpu
from jax.experimental.pallas import tpu_sc as plsc

_NEG_INF = float("-inf")
_I32_MAX = jnp.iinfo(jnp.int32).max


def _top3(cand_v, cand_i, extra_last=0):
    vs, ids = [], []
    for t in range(3):
        m = jnp.max(cand_v, axis=1, keepdims=True)
        sel = jnp.min(jnp.where(cand_v == m, cand_i, _I32_MAX),
                      axis=1, keepdims=True)
        vs.append(m)
        ids.append(sel)
        if t < 2:
            cand_v = jnp.where(cand_i == sel, _NEG_INF, cand_v)
    return vs, ids


def _dist_topk_body(x_ref, e_ref, x2_ref, e2_ref, dist_ref, cv_ref, ci_ref,
                    *, bm, bn, scr):
    j = pl.program_id(1)

    xe = lax.dot_general(
        x_ref[...], e_ref[...],
        dimension_numbers=(((1,), (1,)), ((), ())),
        preferred_element_type=jnp.float32)
    d_tile = -((x2_ref[...] - 2.0 * xe) + e2_ref[...])
    dist_ref[...] = d_tile

    liota = lax.broadcasted_iota(jnp.int32, (1, bn), 1).astype(jnp.float32)
    dd = d_tile
    tvs, tsel = [], []
    for t in range(3):
        m = jnp.max(dd, axis=1, keepdims=True)
        sel = jnp.min(jnp.where(dd == m, liota, jnp.inf),
                      axis=1, keepdims=True)
        tvs.append(m)
        tsel.append(sel)
        if t < 2:
            dd = jnp.where(liota == sel, _NEG_INF, dd)
    tis = [s.astype(jnp.int32) + j * bn for s in tsel]

    pad_v = jnp.full((bm, scr - 3), _NEG_INF, jnp.float32)
    pad_i = jnp.full((bm, scr - 3), _I32_MAX, jnp.int32)
    cv_ref[...] = jnp.concatenate(tvs + [pad_v], axis=1)[None]
    ci_ref[...] = jnp.concatenate(tis + [pad_i], axis=1)[None]


def _merge_body(k_ref, cv_ref, ci_ref, ind_ref, *, nn, scr):
    cv = jnp.concatenate([cv_ref[jj] for jj in range(nn)], axis=1)
    ci = jnp.concatenate([ci_ref[jj] for jj in range(nn)], axis=1)
    _, mis = _top3(cv, ci)
    bm = cv.shape[0]
    kk = k_ref[0]
    ind = jnp.where(kk <= 0, mis[0], jnp.where(kk == 1, mis[1], mis[2]))
    ind_ref[...] = jnp.concatenate(
        [ind, jnp.zeros((bm, scr - 1), jnp.int32)], axis=1)


def _dist_topk(flat, e, x2, e2, k, *, bm=1024, bn=2048, interpret=False):
    bnrows, d = flat.shape
    kk = e.shape[0]
    nm, nn = bnrows // bm, kk // bn
    scr = 8
    body = functools.partial(_dist_topk_body, bm=bm, bn=bn, scr=scr)
    dist2d, cv, ci = pl.pallas_call(
        body,
        grid=(nm, nn),
        in_specs=[
            pl.BlockSpec((bm, d), lambda i, j: (i, 0)),
            pl.BlockSpec((bn, d), lambda i, j: (j, 0)),
            pl.BlockSpec((bm, 1), lambda i, j: (i, 0)),
            pl.BlockSpec((1, bn), lambda i, j: (0, j)),
        ],
        out_specs=[
            pl.BlockSpec((bm, bn), lambda i, j: (i, j)),
            pl.BlockSpec((1, bm, scr), lambda i, j: (j, i, 0)),
            pl.BlockSpec((1, bm, scr), lambda i, j: (j, i, 0)),
        ],
        out_shape=[
            jax.ShapeDtypeStruct((bnrows, kk), jnp.float32),
            jax.ShapeDtypeStruct((nn, bnrows, scr), jnp.float32),
            jax.ShapeDtypeStruct((nn, bnrows, scr), jnp.int32),
        ],
        interpret=interpret,
    )(flat, e, x2, e2)

    mbody = functools.partial(_merge_body, nn=nn, scr=scr)
    k_arr = jnp.asarray(k, jnp.int32).reshape(1)
    ind8 = pl.pallas_call(
        mbody,
        grid=(nm,),
        in_specs=[
            pl.BlockSpec(memory_space=pltpu.SMEM),
            pl.BlockSpec((nn, bm, scr), lambda i: (0, i, 0)),
            pl.BlockSpec((nn, bm, scr), lambda i: (0, i, 0)),
        ],
        out_specs=pl.BlockSpec((bm, scr), lambda i: (i, 0)),
        out_shape=jax.ShapeDtypeStruct((bnrows, scr), jnp.int32),
        interpret=interpret,
    )(k_arr, cv, ci)
    return dist2d, ind8[:, 0]


def _sc_gather(table, idx):
    v, d = table.shape
    b = idx.shape[0]
    info = plsc.get_sparse_core_info()
    nc, ns = info.num_cores, info.num_subcores
    nw = nc * ns
    b_per_w = b // nw
    mesh = plsc.VectorSubcoreMesh(core_axis_name="c", subcore_axis_name="s")

    @functools.partial(
        pl.kernel, mesh=mesh,
        out_type=jax.ShapeDtypeStruct((b, d), jnp.float32),
        scratch_types=[
            pltpu.VMEM((b_per_w,), jnp.int32),
            pltpu.VMEM((b_per_w, d), jnp.float32),
            pltpu.SemaphoreType.DMA,
        ],
    )
    def gather_k(table_hbm, idx_hbm, out_hbm, idx_v, rows_v, sem):
        wid = lax.axis_index("s") * nc + lax.axis_index("c")
        base = wid * b_per_w
        pltpu.sync_copy(idx_hbm.at[pl.ds(base, b_per_w)], idx_v)
        pltpu.async_copy(table_hbm.at[idx_v], rows_v, sem).wait()
        pltpu.sync_copy(rows_v, out_hbm.at[pl.ds(base, b_per_w)])

    return gather_k(table, idx)


def kernel(x, k, embed):
    b, n, d = x.shape
    kk = embed.shape[1]
    flat = x.reshape(b * n, d)
    e = embed[0]

    x2 = jnp.sum(flat ** 2, axis=-1, keepdims=True)
    e2 = jnp.sum(embed ** 2, axis=-1)

    dist2d, ind = _dist_topk(flat, e, x2, e2, k)

    quantize = _sc_gather(e, ind).reshape(b, n, d)
    embed_ind = ind.reshape(b, n)
    dist = dist2d.reshape(1, b * n, kk)
    return quantize, embed_ind, dist
